# baseline (speedup 1.0000x reference)
import functools

import jax
import jax.numpy as jnp
from jax import lax
from jax.experimental import pallas as pl
from jax.experimental.pallas import tpu as pltpu
from jax.experimental.pallas import tpu_sc as plsc

_BATCH = 8
_NUM_ROIS = 100
_RES = 28
_NUM_CLASSES = 91
_ROWS = _BATCH * _NUM_ROIS
_YX = _RES * _RES
_NW = 32
_SC_PER_W = 15
_SC_YX = _NW * _SC_PER_W
_TC_YX = _YX - _SC_YX
_NXB = 4
_L = 16


@functools.partial(
    pl.kernel,
    mesh=plsc.VectorSubcoreMesh(core_axis_name="c", subcore_axis_name="s"),
    out_type=jax.ShapeDtypeStruct((_SC_YX, _BATCH, _NUM_ROIS), jnp.float32),
    scratch_types=[
        pltpu.VMEM((_ROWS,), jnp.int32),
        pltpu.VMEM((_NUM_CLASSES, _BATCH, _NUM_ROIS), jnp.float32),
        pltpu.VMEM((_SC_PER_W, _BATCH, _NUM_ROIS), jnp.float32),
        pltpu.SemaphoreType.DMA,
    ],
    compiler_params=pltpu.CompilerParams(needs_layout_passes=False),
)
def _sc_gather(planes_hbm, cls_hbm, out_hbm, cls_v, stage_v, out_v, sem):
    wid = lax.axis_index("s") * 2 + lax.axis_index("c")
    base = wid * _SC_PER_W
    pltpu.sync_copy(cls_hbm, cls_v)
    iota = lax.iota(jnp.int32, _L)

    def body(j, _):
        pltpu.async_copy(planes_hbm.at[:, _TC_YX + base + j], stage_v,
                         sem).wait()
        jvec = iota * 0 + j
        for k in range(_ROWS // _L):
            pos = iota + (k * _L)
            b0 = (k * _L) // _NUM_ROIS
            bvec = jnp.where(pos >= (b0 + 1) * _NUM_ROIS, b0 + 1, b0)
            rvec = pos - bvec * _NUM_ROIS
            vals = plsc.load_gather(
                stage_v, [cls_v[pl.ds(k * _L, _L)], bvec, rvec])
            plsc.store_scatter(out_v, [jvec, bvec, rvec], vals)
        return 0

    lax.fori_loop(0, _SC_PER_W, body, 0)
    pltpu.sync_copy(out_v, out_hbm.at[pl.ds(base, _SC_PER_W)])


def _tc_body(cls_ref, in_ref, out_ref):
    cls = cls_ref[...]
    acc = in_ref[0]
    for c in range(1, _NUM_CLASSES):
        acc = jnp.where((cls == c)[None], in_ref[c], acc)
    out_ref[...] = acc


def kernel(mask_outputs, class_indices):
    planes = jnp.transpose(mask_outputs, (2, 3, 4, 0, 1)).reshape(
        _NUM_CLASSES, _YX, _BATCH, _NUM_ROIS)
    cls2 = class_indices.astype(jnp.int32)
    cls1 = class_indices.reshape(_ROWS).astype(jnp.int32)
    out_sc = _sc_gather(planes, cls1)
    out_tc = pl.pallas_call(
        _tc_body,
        grid=(_TC_YX // _NXB,),
        in_specs=[
            pl.BlockSpec((_BATCH, _NUM_ROIS), lambda i: (0, 0)),
            pl.BlockSpec((_NUM_CLASSES, _NXB, _BATCH, _NUM_ROIS),
                         lambda i: (0, i, 0, 0)),
        ],
        out_specs=pl.BlockSpec((_NXB, _BATCH, _NUM_ROIS),
                               lambda i: (i, 0, 0)),
        out_shape=jax.ShapeDtypeStruct((_TC_YX, _BATCH, _NUM_ROIS),
                                       jnp.float32),
        compiler_params=pltpu.CompilerParams(
            dimension_semantics=("arbitrary",)),
    )(cls2, planes)
    out = jnp.concatenate([out_tc, out_sc], axis=0)
    return jnp.transpose(out.reshape(_RES, _RES, _BATCH, _NUM_ROIS),
                         (2, 3, 0, 1))


# speedup vs baseline: 1.0233x; 1.0233x over previous
import functools

import jax
import jax.numpy as jnp
from jax import lax
from jax.experimental import pallas as pl
from jax.experimental.pallas import tpu as pltpu
from jax.experimental.pallas import tpu_sc as plsc

_BATCH = 8
_NUM_ROIS = 100
_RES = 28
_NUM_CLASSES = 91
_ROWS = _BATCH * _NUM_ROIS
_YX = _RES * _RES
_NW = 32
_SC_PER_W = 14
_SC_YX = _NW * _SC_PER_W
_TC_YX = _YX - _SC_YX
_NXB = 4
_L = 16


@functools.partial(
    pl.kernel,
    mesh=plsc.VectorSubcoreMesh(core_axis_name="c", subcore_axis_name="s"),
    out_type=jax.ShapeDtypeStruct((_SC_YX, _BATCH, _NUM_ROIS), jnp.float32),
    scratch_types=[
        pltpu.VMEM((_ROWS,), jnp.int32),
        pltpu.VMEM((_NUM_CLASSES, _BATCH, _NUM_ROIS), jnp.float32),
        pltpu.VMEM((_SC_PER_W, _BATCH, _NUM_ROIS), jnp.float32),
        pltpu.SemaphoreType.DMA,
    ],
    compiler_params=pltpu.CompilerParams(needs_layout_passes=False),
)
def _sc_gather(planes_hbm, cls_hbm, out_hbm, cls_v, stage_v, out_v, sem):
    wid = lax.axis_index("s") * 2 + lax.axis_index("c")
    base = wid * _SC_PER_W
    pltpu.sync_copy(cls_hbm, cls_v)
    iota = lax.iota(jnp.int32, _L)

    def body(j, _):
        pltpu.async_copy(planes_hbm.at[:, _TC_YX + base + j], stage_v,
                         sem).wait()
        jvec = iota * 0 + j
        for k in range(_ROWS // _L):
            pos = iota + (k * _L)
            b0 = (k * _L) // _NUM_ROIS
            bvec = jnp.where(pos >= (b0 + 1) * _NUM_ROIS, b0 + 1, b0)
            rvec = pos - bvec * _NUM_ROIS
            vals = plsc.load_gather(
                stage_v, [cls_v[pl.ds(k * _L, _L)], bvec, rvec])
            plsc.store_scatter(out_v, [jvec, bvec, rvec], vals)
        return 0

    lax.fori_loop(0, _SC_PER_W, body, 0)
    pltpu.sync_copy(out_v, out_hbm.at[pl.ds(base, _SC_PER_W)])


def _tc_body(cls_ref, in_ref, out_ref):
    cls = cls_ref[...]
    acc = in_ref[0]
    for c in range(1, _NUM_CLASSES):
        acc = jnp.where((cls == c)[None], in_ref[c], acc)
    out_ref[...] = acc


def kernel(mask_outputs, class_indices):
    planes = jnp.transpose(mask_outputs, (2, 3, 4, 0, 1)).reshape(
        _NUM_CLASSES, _YX, _BATCH, _NUM_ROIS)
    cls2 = class_indices.astype(jnp.int32)
    cls1 = class_indices.reshape(_ROWS).astype(jnp.int32)
    out_sc = _sc_gather(planes, cls1)
    out_tc = pl.pallas_call(
        _tc_body,
        grid=(_TC_YX // _NXB,),
        in_specs=[
            pl.BlockSpec((_BATCH, _NUM_ROIS), lambda i: (0, 0)),
            pl.BlockSpec((_NUM_CLASSES, _NXB, _BATCH, _NUM_ROIS),
                         lambda i: (0, i, 0, 0)),
        ],
        out_specs=pl.BlockSpec((_NXB, _BATCH, _NUM_ROIS),
                               lambda i: (i, 0, 0)),
        out_shape=jax.ShapeDtypeStruct((_TC_YX, _BATCH, _NUM_ROIS),
                                       jnp.float32),
        compiler_params=pltpu.CompilerParams(
            dimension_semantics=("arbitrary",)),
    )(cls2, planes)
    out = jnp.concatenate([out_tc, out_sc], axis=0)
    return jnp.transpose(out.reshape(_RES, _RES, _BATCH, _NUM_ROIS),
                         (2, 3, 0, 1))
